# baseline (speedup 1.0000x reference)
import jax
import jax.numpy as jnp
from jax import lax
from jax.experimental import pallas as pl

B, N, C = 16, 256, 256
BB, CB = 8, 128
CHUNKS = N // 8


def _argmax_one(x_ref, b, o_ref, ob):
    m = x_ref[b, 0:8, :]
    idx = jnp.zeros((8, CB), jnp.int32)
    for c in range(1, CHUNKS):
        v = x_ref[b, 8 * c:8 * c + 8, :]
        pred = v > m
        m = jnp.where(pred, v, m)
        idx = jnp.where(pred, jnp.full((8, CB), c, jnp.int32), idx)
    row = idx * 8 + lax.broadcasted_iota(jnp.int32, (8, CB), 0)
    gmax = jnp.max(m, axis=0, keepdims=True)
    cand = jnp.where(m == gmax, row, N)
    o_ref[ob, :] = jnp.min(cand, axis=0)


def _argmax_body(xa_ref, xb_ref, o_ref):
    for b in range(BB // 2):
        _argmax_one(xa_ref, b, o_ref, b)
    for b in range(BB // 2):
        _argmax_one(xb_ref, b, o_ref, BB // 2 + b)


def kernel(x):
    half = BB // 2
    out = pl.pallas_call(
        _argmax_body,
        grid=(B // BB, C // CB),
        in_specs=[
            pl.BlockSpec((half, N, CB), lambda i, j: (2 * i, 0, j)),
            pl.BlockSpec((half, N, CB), lambda i, j: (2 * i + 1, 0, j)),
        ],
        out_specs=pl.BlockSpec((BB, CB), lambda i, j: (i, j)),
        out_shape=jax.ShapeDtypeStruct((B, C), jnp.int32),
    )(x, x)
    return out.astype(jnp.int64)


# speedup vs baseline: 1.3691x; 1.3691x over previous
import jax
import jax.numpy as jnp
from jax import lax
from jax.experimental import pallas as pl
from jax.experimental.pallas import tpu as pltpu

B, N, C = 16, 256, 256
CHUNKS = N // 8


def _argmax_batch(vbuf, b, o_ref):
    m = vbuf[b, 0:8, :]
    idx = jnp.zeros((8, C), jnp.int32)
    for c in range(1, CHUNKS):
        v = vbuf[b, 8 * c:8 * c + 8, :]
        pred = v > m
        m = jnp.where(pred, v, m)
        idx = jnp.where(pred, jnp.full((8, C), c, jnp.int32), idx)
    row = idx * 8 + lax.broadcasted_iota(jnp.int32, (8, C), 0)
    gmax = jnp.max(m, axis=0, keepdims=True)
    cand = jnp.where(m == gmax, row, N)
    o_ref[b, :] = jnp.min(cand, axis=0)


def _argmax_body(x_hbm, o_ref, vbuf, sems):
    copies = [
        pltpu.make_async_copy(x_hbm.at[b], vbuf.at[b], sems.at[b])
        for b in range(B)
    ]
    for cp in copies:
        cp.start()
    for b in range(B):
        copies[b].wait()
        _argmax_batch(vbuf, b, o_ref)


def kernel(x):
    out = pl.pallas_call(
        _argmax_body,
        in_specs=[pl.BlockSpec(memory_space=pltpu.MemorySpace.HBM)],
        out_specs=pl.BlockSpec(memory_space=pltpu.MemorySpace.VMEM),
        out_shape=jax.ShapeDtypeStruct((B, C), jnp.int32),
        scratch_shapes=[
            pltpu.VMEM((B, N, C), jnp.float32),
            pltpu.SemaphoreType.DMA((B,)),
        ],
    )(x)
    return out.astype(jnp.int64)
